# baseline (speedup 1.0000x reference)
import jax
import jax.numpy as jnp
from jax import lax
from jax.experimental import pallas as pl
from jax.experimental.pallas import tpu as pltpu
from jax.experimental.pallas import tpu_sc as plsc

N = 10000
E = 320000
D = 128
DH = D // 2
NC = 2
NS = 16
NW = NC * NS
CH = 128
NCHUNK = 2560
EPAD = NCHUNK * CH
CPT_A = NCHUNK // NW
CPT_B = NCHUNK // NS
CL = 16
NPAD = 10240
RPT = NPAD // NS
RB = 400


def _count_kernel():
    mesh = plsc.VectorSubcoreMesh(
        core_axis_name="c", subcore_axis_name="s",
        num_cores=NC, num_subcores=NS)

    def body(dsth, zch, och, cnt_o, dst_v, cbuf, ones_v, cnt_s, sem0):
        c = lax.axis_index("c")
        s = lax.axis_index("s")
        wid = s * NC + c

        pltpu.sync_copy(dsth.at[pl.ds(wid * CPT_A, CPT_A)], dst_v)
        pltpu.sync_copy(zch, cbuf)
        pltpu.sync_copy(cbuf, cnt_s.at[pl.ds(s * RPT, RPT)])
        pltpu.sync_copy(och, ones_v)
        plsc.subcore_barrier()

        def chunk(j, carry):
            pltpu.sync_copy(ones_v, cnt_s.at[dst_v.at[j]], add=True)
            return carry

        lax.fori_loop(0, CPT_A, chunk, 0)
        plsc.subcore_barrier()

        pltpu.sync_copy(cnt_s.at[pl.ds(s * RPT, RPT)], cbuf)
        pltpu.sync_copy(cbuf, cnt_o.at[c].at[pl.ds(s * RPT, RPT)])

    return pl.kernel(
        body,
        out_type=[jax.ShapeDtypeStruct((NC, NPAD, CL), jnp.float32)],
        mesh=mesh,
        compiler_params=pltpu.CompilerParams(use_tc_tiling_on_sc=False),
        scratch_types=[
            pltpu.VMEM((CPT_A, CH), jnp.int32),
            pltpu.VMEM((RPT, CL), jnp.float32),
            pltpu.VMEM((CH, CL), jnp.float32),
            pltpu.VMEM_SHARED((NPAD, CL), jnp.float32),
            pltpu.SemaphoreType.DMA,
        ])


def _seg_sum_kernel():
    mesh = plsc.VectorSubcoreMesh(
        core_axis_name="c", subcore_axis_name="s",
        num_cores=NC, num_subcores=NS)

    NB = 5
    LA = 3

    def body(feat, srch, dsth, zfh, agg_o, src_v, dst_v, *rest):
        bufs = tuple(rest[0:NB])
        agg_s = rest[NB]
        gsems = tuple(rest[NB + 1:NB + 1 + NB])
        ssems = tuple(rest[NB + 1 + NB:])
        c = lax.axis_index("c")
        s = lax.axis_index("s")

        pltpu.sync_copy(srch.at[c].at[pl.ds(s * CPT_B, CPT_B)], src_v)
        pltpu.sync_copy(dsth.at[pl.ds(s * CPT_B, CPT_B)], dst_v)

        pltpu.sync_copy(zfh, bufs[0])
        for k in range(RPT // CH):
            pltpu.sync_copy(bufs[0], agg_s.at[pl.ds(s * RPT + k * CH, CH)])
        plsc.subcore_barrier()

        for j0 in range(LA):
            pltpu.async_copy(feat.at[src_v.at[j0]], bufs[j0], gsems[j0])

        def step(p, carry):
            for b in range(NB):
                j = NB * p + b
                buf = bufs[b]
                pltpu.make_async_copy(
                    feat.at[src_v.at[j]], buf, gsems[b]).wait()
                pltpu.async_copy(
                    buf, agg_s.at[dst_v.at[j]], ssems[b], add=True)
                bn = (b + LA) % NB
                jw = j + LA - NB

                @pl.when(jw >= 0)
                def _():
                    pltpu.make_async_copy(
                        bufs[bn], agg_s.at[dst_v.at[jw]], ssems[bn]).wait()

                jn = j + LA

                @pl.when(jn < CPT_B)
                def _():
                    pltpu.async_copy(
                        feat.at[src_v.at[jn]], bufs[bn], gsems[bn])
            return carry

        lax.fori_loop(0, CPT_B // NB, step, 0)
        for j in range(CPT_B - CPT_B % NB, CPT_B):
            b = j % NB
            pltpu.make_async_copy(
                feat.at[src_v.at[j]], bufs[b], gsems[b]).wait()
            pltpu.async_copy(
                bufs[b], agg_s.at[dst_v.at[j]], ssems[b], add=True)
            jw = j + LA - NB
            bw = jw % NB
            pltpu.make_async_copy(
                bufs[bw], agg_s.at[dst_v.at[jw]], ssems[bw]).wait()
            jn = j + LA
            if jn < CPT_B:
                pltpu.async_copy(feat.at[src_v.at[jn]], bufs[jn % NB],
                                 gsems[jn % NB])
        for j in range(CPT_B - (NB - LA), CPT_B):
            b = j % NB
            pltpu.make_async_copy(
                bufs[b], agg_s.at[dst_v.at[j]], ssems[b]).wait()
        plsc.subcore_barrier()

        for k in range(RPT // CH):
            r0 = s * RPT + k * CH
            pltpu.sync_copy(agg_s.at[pl.ds(r0, CH)], bufs[0])
            pltpu.sync_copy(bufs[0], agg_o.at[c].at[pl.ds(r0, CH)])

    return pl.kernel(
        body,
        out_type=[jax.ShapeDtypeStruct((NC, NPAD, DH), jnp.float32)],
        mesh=mesh,
        compiler_params=pltpu.CompilerParams(use_tc_tiling_on_sc=False),
        scratch_types=[
            pltpu.VMEM((CPT_B, CH), jnp.int32),
            pltpu.VMEM((CPT_B, CH), jnp.int32),
        ] + [pltpu.VMEM((CH, DH), jnp.float32)] * NB + [
            pltpu.VMEM_SHARED((NPAD, DH), jnp.float32),
        ] + [pltpu.SemaphoreType.DMA] * (2 * NB))


def _make_dense(relu: bool):
    def body(agg_ref, cnt_ref, x_ref, wl_ref, wr_ref, b_ref, o_ref):
        cnt = jnp.maximum(cnt_ref[0, :, 0:1] + cnt_ref[1, :, 0:1], 1.0)
        acc = jnp.dot(agg_ref[0] / cnt, wl_ref[0:DH, :],
                      preferred_element_type=jnp.float32)
        acc = acc + jnp.dot(agg_ref[1] / cnt, wl_ref[DH:D, :],
                            preferred_element_type=jnp.float32)
        acc = acc + jnp.dot(x_ref[...], wr_ref[...],
                            preferred_element_type=jnp.float32)
        acc = acc + b_ref[...]
        if relu:
            acc = jnp.maximum(acc, 0.0)
        o_ref[...] = acc

    return pl.pallas_call(
        body,
        grid=(N // RB,),
        in_specs=[
            pl.BlockSpec((NC, RB, DH), lambda i: (0, i, 0)),
            pl.BlockSpec((NC, RB, CL), lambda i: (0, i, 0)),
            pl.BlockSpec((RB, D), lambda i: (i, 0)),
            pl.BlockSpec((D, D), lambda i: (0, 0)),
            pl.BlockSpec((D, D), lambda i: (0, 0)),
            pl.BlockSpec((1, D), lambda i: (0, 0)),
        ],
        out_specs=pl.BlockSpec((RB, D), lambda i: (i, 0)),
        out_shape=jax.ShapeDtypeStruct((N, D), jnp.float32),
    )


_count = _count_kernel()
_seg_sum = _seg_sum_kernel()
_dense_relu = _make_dense(relu=True)
_dense = _make_dense(relu=False)


def kernel(x, edge_index, Wl1, Wr1, b1, Wl2, Wr2, b2):
    src = edge_index[0].astype(jnp.int32)
    dst = edge_index[1].astype(jnp.int32)
    pad = EPAD - E
    src2d = jnp.concatenate(
        [src, jnp.zeros((pad,), jnp.int32)]).reshape(NCHUNK, CH)
    dst2d = jnp.concatenate(
        [dst, jnp.full((pad,), N, jnp.int32)]).reshape(NCHUNK, CH)
    zf = jnp.zeros((CH, DH), jnp.float32)
    zc = jnp.zeros((RPT, CL), jnp.float32)
    oc = jnp.ones((CH, CL), jnp.float32)
    b1r = b1.reshape(1, D)
    b2r = b2.reshape(1, D)

    src2dc = jnp.stack([src2d, src2d + N])

    cnt, = _count(dst2d, zc, oc)
    x2 = jnp.concatenate([x[:, :DH], x[:, DH:]], axis=0)
    agg1, = _seg_sum(x2, src2dc, dst2d, zf)
    h = _dense_relu(agg1, cnt, x, Wl1, Wr1, b1r)
    h2 = jnp.concatenate([h[:, :DH], h[:, DH:]], axis=0)
    agg2, = _seg_sum(h2, src2dc, dst2d, zf)
    out = _dense(agg2, cnt, h, Wl2, Wr2, b2r)
    return out


# speedup vs baseline: 1.0297x; 1.0297x over previous
import jax
import jax.numpy as jnp
from jax import lax
from jax.experimental import pallas as pl
from jax.experimental.pallas import tpu as pltpu
from jax.experimental.pallas import tpu_sc as plsc

N = 10000
E = 320000
D = 128
DH = D // 2
NC = 2
NS = 16
NW = NC * NS
CH = 128
NCHUNK = 2560
EPAD = NCHUNK * CH
CPT_A = NCHUNK // NW
CPT_B = NCHUNK // NS
CL = 16
NPAD = 10240
RPT = NPAD // NS
RB = 400


def _count_kernel():
    mesh = plsc.VectorSubcoreMesh(
        core_axis_name="c", subcore_axis_name="s",
        num_cores=NC, num_subcores=NS)

    def body(dsth, zch, och, cnt_o, dst_v, cbuf, ones_v, cnt_s, sem0):
        c = lax.axis_index("c")
        s = lax.axis_index("s")
        wid = s * NC + c

        pltpu.sync_copy(dsth.at[pl.ds(wid * CPT_A, CPT_A)], dst_v)
        pltpu.sync_copy(zch, cbuf)
        pltpu.sync_copy(cbuf, cnt_s.at[pl.ds(s * RPT, RPT)])
        pltpu.sync_copy(och, ones_v)
        plsc.subcore_barrier()

        def chunk(j, carry):
            pltpu.sync_copy(ones_v, cnt_s.at[dst_v.at[j]], add=True)
            return carry

        lax.fori_loop(0, CPT_A, chunk, 0)
        plsc.subcore_barrier()

        pltpu.sync_copy(cnt_s.at[pl.ds(s * RPT, RPT)], cbuf)
        pltpu.sync_copy(cbuf, cnt_o.at[c].at[pl.ds(s * RPT, RPT)])

    return pl.kernel(
        body,
        out_type=[jax.ShapeDtypeStruct((NC, NPAD, CL), jnp.float32)],
        mesh=mesh,
        compiler_params=pltpu.CompilerParams(use_tc_tiling_on_sc=False),
        scratch_types=[
            pltpu.VMEM((CPT_A, CH), jnp.int32),
            pltpu.VMEM((RPT, CL), jnp.float32),
            pltpu.VMEM((CH, CL), jnp.float32),
            pltpu.VMEM_SHARED((NPAD, CL), jnp.float32),
            pltpu.SemaphoreType.DMA,
        ])


def _seg_sum_kernel():
    mesh = plsc.VectorSubcoreMesh(
        core_axis_name="c", subcore_axis_name="s",
        num_cores=NC, num_subcores=NS)

    NB = 5
    LA = 3

    def body(feat, srch, dsth, zfh, agg_o, src_v, dst_v, *rest):
        bufs = tuple(rest[0:NB])
        agg_s = rest[NB]
        gsems = tuple(rest[NB + 1:NB + 1 + NB])
        ssems = tuple(rest[NB + 1 + NB:])
        c = lax.axis_index("c")
        s = lax.axis_index("s")

        pltpu.sync_copy(srch.at[c].at[pl.ds(s * CPT_B, CPT_B)], src_v)
        pltpu.sync_copy(dsth.at[pl.ds(s * CPT_B, CPT_B)], dst_v)

        pltpu.sync_copy(zfh, bufs[0])
        for k in range(RPT // CH):
            pltpu.sync_copy(bufs[0], agg_s.at[pl.ds(s * RPT + k * CH, CH)])
        plsc.subcore_barrier()

        for j0 in range(LA):
            pltpu.async_copy(feat.at[src_v.at[j0]], bufs[j0], gsems[j0])

        def step(p, carry):
            for b in range(NB):
                j = NB * p + b
                buf = bufs[b]
                pltpu.make_async_copy(
                    feat.at[src_v.at[j]], buf, gsems[b]).wait()
                pltpu.async_copy(
                    buf, agg_s.at[dst_v.at[j]], ssems[b], add=True)
                bn = (b + LA) % NB
                jw = j + LA - NB

                @pl.when(jw >= 0)
                def _():
                    pltpu.make_async_copy(
                        bufs[bn], agg_s.at[dst_v.at[jw]], ssems[bn]).wait()

                jn = j + LA

                @pl.when(jn < CPT_B)
                def _():
                    pltpu.async_copy(
                        feat.at[src_v.at[jn]], bufs[bn], gsems[bn])
            return carry

        lax.fori_loop(0, CPT_B // NB, step, 0)
        for j in range(CPT_B - CPT_B % NB, CPT_B):
            b = j % NB
            pltpu.make_async_copy(
                feat.at[src_v.at[j]], bufs[b], gsems[b]).wait()
            pltpu.async_copy(
                bufs[b], agg_s.at[dst_v.at[j]], ssems[b], add=True)
            jw = j + LA - NB
            bw = jw % NB
            pltpu.make_async_copy(
                bufs[bw], agg_s.at[dst_v.at[jw]], ssems[bw]).wait()
            jn = j + LA
            if jn < CPT_B:
                pltpu.async_copy(feat.at[src_v.at[jn]], bufs[jn % NB],
                                 gsems[jn % NB])
        for j in range(CPT_B - (NB - LA), CPT_B):
            b = j % NB
            pltpu.make_async_copy(
                bufs[b], agg_s.at[dst_v.at[j]], ssems[b]).wait()
        plsc.subcore_barrier()

        for k in range(RPT // CH):
            r0 = s * RPT + k * CH
            pltpu.sync_copy(agg_s.at[pl.ds(r0, CH)], bufs[0])
            pltpu.sync_copy(bufs[0], agg_o.at[c].at[pl.ds(r0, CH)])

    return pl.kernel(
        body,
        out_type=[jax.ShapeDtypeStruct((NC, NPAD, DH), jnp.float32)],
        mesh=mesh,
        compiler_params=pltpu.CompilerParams(use_tc_tiling_on_sc=False),
        scratch_types=[
            pltpu.VMEM((CPT_B, CH), jnp.int32),
            pltpu.VMEM((CPT_B, CH), jnp.int32),
        ] + [pltpu.VMEM((CH, DH), jnp.float32)] * NB + [
            pltpu.VMEM_SHARED((NPAD, DH), jnp.float32),
        ] + [pltpu.SemaphoreType.DMA] * (2 * NB))


def _make_dense(relu: bool):
    def body(agg_ref, cnt_ref, x_ref, wl_ref, wr_ref, b_ref, o_ref,
             o2_ref=None):
        cnt = jnp.maximum(cnt_ref[0, :, 0:1] + cnt_ref[1, :, 0:1], 1.0)
        acc = jnp.dot(agg_ref[0] / cnt, wl_ref[0:DH, :],
                      preferred_element_type=jnp.float32)
        acc = acc + jnp.dot(agg_ref[1] / cnt, wl_ref[DH:D, :],
                            preferred_element_type=jnp.float32)
        acc = acc + jnp.dot(x_ref[...], wr_ref[...],
                            preferred_element_type=jnp.float32)
        acc = acc + b_ref[...]
        if relu:
            acc = jnp.maximum(acc, 0.0)
        o_ref[...] = acc
        if o2_ref is not None:
            o2_ref[0] = acc[:, 0:DH]
            o2_ref[1] = acc[:, DH:D]

    out_specs = pl.BlockSpec((RB, D), lambda i: (i, 0))
    out_shape = jax.ShapeDtypeStruct((N, D), jnp.float32)
    if relu:
        out_specs = [out_specs, pl.BlockSpec((NC, RB, DH),
                                             lambda i: (0, i, 0))]
        out_shape = [out_shape,
                     jax.ShapeDtypeStruct((NC, N, DH), jnp.float32)]

    return pl.pallas_call(
        body,
        grid=(N // RB,),
        in_specs=[
            pl.BlockSpec((NC, RB, DH), lambda i: (0, i, 0)),
            pl.BlockSpec((NC, RB, CL), lambda i: (0, i, 0)),
            pl.BlockSpec((RB, D), lambda i: (i, 0)),
            pl.BlockSpec((D, D), lambda i: (0, 0)),
            pl.BlockSpec((D, D), lambda i: (0, 0)),
            pl.BlockSpec((1, D), lambda i: (0, 0)),
        ],
        out_specs=out_specs,
        out_shape=out_shape,
    )


_count = _count_kernel()
_seg_sum = _seg_sum_kernel()
_dense_relu = _make_dense(relu=True)
_dense = _make_dense(relu=False)


def kernel(x, edge_index, Wl1, Wr1, b1, Wl2, Wr2, b2):
    src = edge_index[0].astype(jnp.int32)
    dst = edge_index[1].astype(jnp.int32)
    pad = EPAD - E
    src2d = jnp.concatenate(
        [src, jnp.zeros((pad,), jnp.int32)]).reshape(NCHUNK, CH)
    dst2d = jnp.concatenate(
        [dst, jnp.full((pad,), N, jnp.int32)]).reshape(NCHUNK, CH)
    zf = jnp.zeros((CH, DH), jnp.float32)
    zc = jnp.zeros((RPT, CL), jnp.float32)
    oc = jnp.ones((CH, CL), jnp.float32)
    b1r = b1.reshape(1, D)
    b2r = b2.reshape(1, D)

    src2dc = jnp.stack([src2d, src2d + N])

    cnt, = _count(dst2d, zc, oc)
    x2 = jnp.concatenate([x[:, :DH], x[:, DH:]], axis=0)
    agg1, = _seg_sum(x2, src2dc, dst2d, zf)
    h, h2c = _dense_relu(agg1, cnt, x, Wl1, Wr1, b1r)
    agg2, = _seg_sum(h2c.reshape(NC * N, DH), src2dc, dst2d, zf)
    out = _dense(agg2, cnt, h, Wl2, Wr2, b2r)
    return out


# speedup vs baseline: 1.0849x; 1.0536x over previous
import jax
import jax.numpy as jnp
from jax import lax
from jax.experimental import pallas as pl
from jax.experimental.pallas import tpu as pltpu
from jax.experimental.pallas import tpu_sc as plsc

N = 10000
E = 320000
D = 128
DH = D // 2
NC = 2
NS = 16
NW = NC * NS
CH = 128
NCHUNK = 2560
EPAD = NCHUNK * CH
CPT_A = NCHUNK // NW
CPT_B = NCHUNK // NS
CL = 16
NPAD = 10240
RPT = NPAD // NS
RB = 400


def _count_kernel():
    mesh = plsc.VectorSubcoreMesh(
        core_axis_name="c", subcore_axis_name="s",
        num_cores=NC, num_subcores=NS)

    def body(dsth, zch, och, cnt_o, dst_v, cbuf, ones_v, cnt_s, sem0):
        c = lax.axis_index("c")
        s = lax.axis_index("s")
        wid = s * NC + c

        pltpu.sync_copy(dsth.at[pl.ds(wid * CPT_A, CPT_A)], dst_v)
        pltpu.sync_copy(zch, cbuf)
        pltpu.sync_copy(cbuf, cnt_s.at[pl.ds(s * RPT, RPT)])
        pltpu.sync_copy(och, ones_v)
        plsc.subcore_barrier()

        def wave(w, carry):
            for b in range(4):
                pltpu.async_copy(
                    ones_v, cnt_s.at[dst_v.at[w * 4 + b]], sem0, add=True)
            for b in range(4):
                pltpu.make_async_copy(
                    ones_v, cnt_s.at[dst_v.at[w * 4 + b]], sem0).wait()
            return carry

        lax.fori_loop(0, CPT_A // 4, wave, 0)
        plsc.subcore_barrier()

        pltpu.sync_copy(cnt_s.at[pl.ds(s * RPT, RPT)], cbuf)
        pltpu.sync_copy(cbuf, cnt_o.at[c].at[pl.ds(s * RPT, RPT)])

    return pl.kernel(
        body,
        out_type=[jax.ShapeDtypeStruct((NC, NPAD, CL), jnp.float32)],
        mesh=mesh,
        compiler_params=pltpu.CompilerParams(use_tc_tiling_on_sc=False),
        scratch_types=[
            pltpu.VMEM((CPT_A, CH), jnp.int32),
            pltpu.VMEM((RPT, CL), jnp.float32),
            pltpu.VMEM((CH, CL), jnp.float32),
            pltpu.VMEM_SHARED((NPAD, CL), jnp.float32),
            pltpu.SemaphoreType.DMA,
        ])


def _seg_sum_kernel():
    mesh = plsc.VectorSubcoreMesh(
        core_axis_name="c", subcore_axis_name="s",
        num_cores=NC, num_subcores=NS)

    NB = 5
    LA = 3

    def body(feat, srch, dsth, zfh, agg_o, src_v, dst_v, *rest):
        bufs = tuple(rest[0:NB])
        agg_s = rest[NB]
        gsems = tuple(rest[NB + 1:NB + 1 + NB])
        ssems = tuple(rest[NB + 1 + NB:])
        c = lax.axis_index("c")
        s = lax.axis_index("s")

        pltpu.sync_copy(srch.at[c].at[pl.ds(s * CPT_B, CPT_B)], src_v)
        pltpu.sync_copy(dsth.at[pl.ds(s * CPT_B, CPT_B)], dst_v)

        pltpu.sync_copy(zfh, bufs[0])
        for k in range(RPT // CH):
            pltpu.sync_copy(bufs[0], agg_s.at[pl.ds(s * RPT + k * CH, CH)])
        plsc.subcore_barrier()

        for j0 in range(LA):
            pltpu.async_copy(feat.at[src_v.at[j0]], bufs[j0], gsems[j0])

        def step(p, carry):
            for b in range(NB):
                j = NB * p + b
                buf = bufs[b]
                pltpu.make_async_copy(
                    feat.at[src_v.at[j]], buf, gsems[b]).wait()
                pltpu.async_copy(
                    buf, agg_s.at[dst_v.at[j]], ssems[b], add=True)
                bn = (b + LA) % NB
                jw = j + LA - NB

                @pl.when(jw >= 0)
                def _():
                    pltpu.make_async_copy(
                        bufs[bn], agg_s.at[dst_v.at[jw]], ssems[bn]).wait()

                jn = j + LA

                @pl.when(jn < CPT_B)
                def _():
                    pltpu.async_copy(
                        feat.at[src_v.at[jn]], bufs[bn], gsems[bn])
            return carry

        lax.fori_loop(0, CPT_B // NB, step, 0)
        for j in range(CPT_B - CPT_B % NB, CPT_B):
            b = j % NB
            pltpu.make_async_copy(
                feat.at[src_v.at[j]], bufs[b], gsems[b]).wait()
            pltpu.async_copy(
                bufs[b], agg_s.at[dst_v.at[j]], ssems[b], add=True)
            jw = j + LA - NB
            bw = jw % NB
            pltpu.make_async_copy(
                bufs[bw], agg_s.at[dst_v.at[jw]], ssems[bw]).wait()
            jn = j + LA
            if jn < CPT_B:
                pltpu.async_copy(feat.at[src_v.at[jn]], bufs[jn % NB],
                                 gsems[jn % NB])
        for j in range(CPT_B - (NB - LA), CPT_B):
            b = j % NB
            pltpu.make_async_copy(
                bufs[b], agg_s.at[dst_v.at[j]], ssems[b]).wait()
        plsc.subcore_barrier()

        for k in range(RPT // CH):
            r0 = s * RPT + k * CH
            pltpu.sync_copy(agg_s.at[pl.ds(r0, CH)], bufs[0])
            pltpu.sync_copy(bufs[0], agg_o.at[c].at[pl.ds(r0, CH)])

    return pl.kernel(
        body,
        out_type=[jax.ShapeDtypeStruct((NC, NPAD, DH), jnp.float32)],
        mesh=mesh,
        compiler_params=pltpu.CompilerParams(use_tc_tiling_on_sc=False),
        scratch_types=[
            pltpu.VMEM((CPT_B, CH), jnp.int32),
            pltpu.VMEM((CPT_B, CH), jnp.int32),
        ] + [pltpu.VMEM((CH, DH), jnp.float32)] * NB + [
            pltpu.VMEM_SHARED((NPAD, DH), jnp.float32),
        ] + [pltpu.SemaphoreType.DMA] * (2 * NB))


def _make_dense(relu: bool):
    def body(agg_ref, cnt_ref, x_ref, wl_ref, wr_ref, b_ref, o_ref,
             o2_ref=None):
        cnt = jnp.maximum(cnt_ref[0, :, 0:1] + cnt_ref[1, :, 0:1], 1.0)
        acc = jnp.dot(agg_ref[0] / cnt, wl_ref[0:DH, :],
                      preferred_element_type=jnp.float32)
        acc = acc + jnp.dot(agg_ref[1] / cnt, wl_ref[DH:D, :],
                            preferred_element_type=jnp.float32)
        acc = acc + jnp.dot(x_ref[...], wr_ref[...],
                            preferred_element_type=jnp.float32)
        acc = acc + b_ref[...]
        if relu:
            acc = jnp.maximum(acc, 0.0)
        o_ref[...] = acc
        if o2_ref is not None:
            o2_ref[0] = acc[:, 0:DH]
            o2_ref[1] = acc[:, DH:D]

    out_specs = pl.BlockSpec((RB, D), lambda i: (i, 0))
    out_shape = jax.ShapeDtypeStruct((N, D), jnp.float32)
    if relu:
        out_specs = [out_specs, pl.BlockSpec((NC, RB, DH),
                                             lambda i: (0, i, 0))]
        out_shape = [out_shape,
                     jax.ShapeDtypeStruct((NC, N, DH), jnp.float32)]

    return pl.pallas_call(
        body,
        grid=(N // RB,),
        in_specs=[
            pl.BlockSpec((NC, RB, DH), lambda i: (0, i, 0)),
            pl.BlockSpec((NC, RB, CL), lambda i: (0, i, 0)),
            pl.BlockSpec((RB, D), lambda i: (i, 0)),
            pl.BlockSpec((D, D), lambda i: (0, 0)),
            pl.BlockSpec((D, D), lambda i: (0, 0)),
            pl.BlockSpec((1, D), lambda i: (0, 0)),
        ],
        out_specs=out_specs,
        out_shape=out_shape,
    )


_count = _count_kernel()
_seg_sum = _seg_sum_kernel()
_dense_relu = _make_dense(relu=True)
_dense = _make_dense(relu=False)


def kernel(x, edge_index, Wl1, Wr1, b1, Wl2, Wr2, b2):
    src = edge_index[0].astype(jnp.int32)
    dst = edge_index[1].astype(jnp.int32)
    pad = EPAD - E
    src2d = jnp.concatenate(
        [src, jnp.zeros((pad,), jnp.int32)]).reshape(NCHUNK, CH)
    dst2d = jnp.concatenate(
        [dst, jnp.full((pad,), N, jnp.int32)]).reshape(NCHUNK, CH)
    zf = jnp.zeros((CH, DH), jnp.float32)
    zc = jnp.zeros((RPT, CL), jnp.float32)
    oc = jnp.ones((CH, CL), jnp.float32)
    b1r = b1.reshape(1, D)
    b2r = b2.reshape(1, D)

    src2dc = jnp.stack([src2d, src2d + N])

    cnt, = _count(dst2d, zc, oc)
    x2 = jnp.concatenate([x[:, :DH], x[:, DH:]], axis=0)
    agg1, = _seg_sum(x2, src2dc, dst2d, zf)
    h, h2c = _dense_relu(agg1, cnt, x, Wl1, Wr1, b1r)
    agg2, = _seg_sum(h2c.reshape(NC * N, DH), src2dc, dst2d, zf)
    out = _dense(agg2, cnt, h, Wl2, Wr2, b2r)
    return out
